# Initial kernel scaffold; baseline (speedup 1.0000x reference)
import jax
import jax.numpy as jnp
from jax.experimental import pallas as pl


def kernel(x, W, b):
    raise NotImplementedError("write your pallas kernel here")


# speedup vs baseline: 5.4288x; 5.4288x over previous
import jax
import jax.numpy as jnp
from jax.experimental import pallas as pl

E = 64
D = 768
TB = 1024


def _tc_router_body(x_ref, w_ref, b_ref, rw_ref, idx_ref):
    xb = x_ref[...]
    w = w_ref[...]
    logits = jax.lax.dot_general(
        xb, w, (((1,), (1,)), ((), ())),
        preferred_element_type=jnp.float32) + b_ref[...]
    iota = jax.lax.broadcasted_iota(jnp.int32, (TB, E), 1)
    m1 = jnp.max(logits, axis=1, keepdims=True)
    i1 = jnp.min(jnp.where(logits == m1, iota, E), axis=1, keepdims=True)
    masked = jnp.where(iota == i1, -jnp.inf, logits)
    m2 = jnp.max(masked, axis=1, keepdims=True)
    i2 = jnp.min(jnp.where(masked == m2, iota, E), axis=1, keepdims=True)
    ew = jnp.exp(m2 - m1)
    s = 1.0 / (1.0 + ew)
    w1 = s
    w2 = ew * s
    rw_ref[...] = (jnp.where(iota == i1, w1, 0.0)
                   + jnp.where(iota == i2, w2, 0.0))
    iota2 = jax.lax.broadcasted_iota(jnp.int32, (TB, 2), 1)
    idx_ref[...] = jnp.where(iota2 == 0, i1, i2)


def kernel(x, W, b):
    T = x.shape[0] * x.shape[1]
    xf = x.reshape(T, D)
    b2 = b.reshape(1, E)
    grid = (T // TB,)
    rw, idx = pl.pallas_call(
        _tc_router_body,
        grid=grid,
        in_specs=[
            pl.BlockSpec((TB, D), lambda i: (i, 0)),
            pl.BlockSpec((E, D), lambda i: (0, 0)),
            pl.BlockSpec((1, E), lambda i: (0, 0)),
        ],
        out_specs=[
            pl.BlockSpec((TB, E), lambda i: (i, 0)),
            pl.BlockSpec((TB, 2), lambda i: (i, 0)),
        ],
        out_shape=[
            jax.ShapeDtypeStruct((T, E), jnp.float32),
            jax.ShapeDtypeStruct((T, 2), jnp.int32),
        ],
    )(xf, W, b2)
    return (rw, idx)
